# baseline (speedup 1.0000x reference)
import functools

import jax
import jax.numpy as jnp
from jax import lax
from jax.experimental import pallas as pl
from jax.experimental.pallas import tpu as pltpu
from jax.experimental.pallas import tpu_sc as plsc

B = 16384
D = 4096
NW = 32
BPW = B // NW
W = 8
NCHUNK = BPW // W

_mesh = plsc.VectorSubcoreMesh(core_axis_name="c", subcore_axis_name="s")


@functools.partial(
    pl.kernel,
    mesh=_mesh,
    out_type=jax.ShapeDtypeStruct((B, D), jnp.float32),
    scratch_types=[
        pltpu.VMEM((NCHUNK, W), jnp.int32),
        pltpu.VMEM((W, D), jnp.float32),
        pltpu.VMEM((W, D), jnp.float32),
        pltpu.VMEM((W, D), jnp.float32),
        pltpu.SemaphoreType.DMA,
        pltpu.SemaphoreType.DMA,
        pltpu.SemaphoreType.DMA,
    ],
)
def _sc_gather(idx_hbm, table_hbm, out_hbm, idx_v, row0, row1, row2,
               semg0, semg1, semg2):
    bufs = (row0, row1, row2)
    semg = (semg0, semg1, semg2)
    wid = lax.axis_index("s") * 2 + lax.axis_index("c")
    base = wid * BPW
    pltpu.sync_copy(idx_hbm.at[wid], idx_v)

    def gather(c, b):
        pltpu.async_copy(table_hbm.at[idx_v.at[c]], bufs[b], semg[b])

    def wait_gather(c, b):
        pltpu.make_async_copy(table_hbm.at[idx_v.at[c]], bufs[b], semg[b]).wait()

    def write_sync(c, b):
        pltpu.sync_copy(bufs[b], out_hbm.at[pl.ds(base + c * W, W)])

    gather(0, 0)
    gather(1, 1)
    gather(2, 2)

    nf = NCHUNK // 3 - 1

    def body(c3, carry):
        for b in range(3):
            cb = c3 * 3 + b
            wait_gather(cb, b)
            write_sync(cb, b)
            gather(cb + 3, b)
        return carry

    lax.fori_loop(0, nf, body, 0)

    for cb in range(3 * nf, NCHUNK):
        b = cb % 3
        wait_gather(cb, b)
        write_sync(cb, b)
        if cb + 3 < NCHUNK:
            gather(cb + 3, b)


def kernel(positions, weight):
    shape = positions.shape
    idx = positions.reshape(NW, NCHUNK, W).astype(jnp.int32)
    out = _sc_gather(idx, weight)
    return out.reshape(*shape, D)
